# baseline (speedup 1.0000x reference)
import functools

import jax
import jax.numpy as jnp
import numpy as np
from jax import lax
from jax.experimental import pallas as pl
from jax.experimental.pallas import tpu as pltpu
from jax.experimental.pallas import tpu_sc as plsc


def _logmap0(x):
    norm = jnp.sqrt(jnp.sum(x * x, axis=-1, keepdims=True))
    norm_c = jnp.maximum(norm, 1e-10)
    arg = jnp.minimum(norm_c, 1.0 - 1e-5)
    atanh = 0.5 * jnp.log((1.0 + arg) / (1.0 - arg))
    return x * (atanh / norm_c)


def _expmap0(u):
    norm = jnp.sqrt(jnp.sum(u * u, axis=-1, keepdims=True))
    norm_c = jnp.maximum(norm, 1e-10)
    return jnp.tanh(norm_c) * u / norm_c


_ATANH_MAX = float(np.arctanh(np.float64(1.0 - 1e-5)))


def _norm_clip(u):
    norm = jnp.sqrt(jnp.sum(u * u, axis=-1, keepdims=True))
    norm_c = jnp.maximum(norm, 1e-10)
    return u * (jnp.minimum(norm, _ATANH_MAX) / norm_c)


_N = 10000
_D = 128
_ROWS_PER_BLOCK = 10000
_N_BLOCKS = _N // _ROWS_PER_BLOCK


def _tc_pre_body(x_ref, o_ref):
    o_ref[...] = _logmap0(x_ref[...])


_tc_pre = pl.pallas_call(
    _tc_pre_body,
    grid=(_N_BLOCKS,),
    in_specs=[pl.BlockSpec((_ROWS_PER_BLOCK, _D), lambda i: (i, 0))],
    out_specs=pl.BlockSpec((_ROWS_PER_BLOCK, _D), lambda i: (i, 0)),
    out_shape=jax.ShapeDtypeStruct((_N, _D), jnp.float32),
)


def _tc_post_body(p0_ref, p1_ref, t_ref, w_ref, b_ref, o_ref, *, mode):
    i = pl.program_id(0)
    rows = i * _ROWS_PER_BLOCK + lax.broadcasted_iota(
        jnp.int32, (_ROWS_PER_BLOCK, 1), 0)
    pad_fix = jnp.where(rows < _PAD_EDGES, t_ref[...], 0.0)
    s = p0_ref[...] + p1_ref[...] - pad_fix
    o = lax.dot_general(
        s, w_ref[...], (((1,), (1,)), ((), ())),
        preferred_element_type=jnp.float32,
    ) + b_ref[...]
    if mode == "mid":
        h = _norm_clip(jnp.maximum(_norm_clip(o), 0.0))
    else:
        h = _expmap0(o)
    o_ref[...] = h


def _make_tc_post(mode):
    return pl.pallas_call(
        functools.partial(_tc_post_body, mode=mode),
        grid=(_N_BLOCKS,),
        in_specs=[
            pl.BlockSpec((_ROWS_PER_BLOCK, _D), lambda i: (i, 0)),
            pl.BlockSpec((_ROWS_PER_BLOCK, _D), lambda i: (i + _N_BLOCKS, 0)),
            pl.BlockSpec((_ROWS_PER_BLOCK, _D), lambda i: (i, 0)),
            pl.BlockSpec((_D, _D), lambda i: (0, 0)),
            pl.BlockSpec((1, _D), lambda i: (0, 0)),
        ],
        out_specs=pl.BlockSpec((_ROWS_PER_BLOCK, _D), lambda i: (i, 0)),
        out_shape=jax.ShapeDtypeStruct((_N, _D), jnp.float32),
    )


_tc_post_mid = _make_tc_post("mid")
_tc_post_final = _make_tc_post("final")


_NC = 2
_NS = 16
_NW = _NC * _NS
_CHUNK = 120
_GROUP = 3
_GROUPS_PER_W = 28
_CHUNKS_PER_W = _GROUPS_PER_W * _GROUP
_E_PAD = _NW * _CHUNKS_PER_W * _CHUNK
_N_GROUPS = _NW * _GROUPS_PER_W
_PAD_EDGES = _E_PAD - 320000
_ROWS_PER_TILE_OUT = 624

_sc_mesh = plsc.VectorSubcoreMesh(core_axis_name="c", subcore_axis_name="s")


@functools.partial(
    pl.kernel,
    out_type=jax.ShapeDtypeStruct((_NC * _N, _D), jnp.float32),
    mesh=_sc_mesh,
    scratch_types=[
        pltpu.VMEM((2, _GROUP, _CHUNK), jnp.int32),
        pltpu.VMEM((2, _GROUP, _CHUNK), jnp.int32),
        pltpu.VMEM((_GROUP, _CHUNK, _D), jnp.float32),
        pltpu.VMEM_SHARED((_N, _D), jnp.float32),
        pltpu.SemaphoreType.DMA,
        pltpu.SemaphoreType.DMA,
        pltpu.SemaphoreType.DMA,
    ],
)
def _sc_segsum(y_hbm, src_hbm, dst_hbm, zeros_hbm, out_hbm,
               src_idx, dst_idx, bufs, acc, gsem, ssem, isem):
    c = lax.axis_index("c")
    s = lax.axis_index("s")
    wid = c * _NS + s
    gbase = wid * _GROUPS_PER_W

    @pl.when(s < _NS - 1)
    def _():
        pltpu.sync_copy(zeros_hbm.at[pl.ds(s * 632, 632)],
                        acc.at[pl.ds(s * 632, 632)])

    @pl.when(s == _NS - 1)
    def _():
        pltpu.sync_copy(zeros_hbm.at[pl.ds(9480, 520)],
                        acc.at[pl.ds(9480, 520)])

    plsc.subcore_barrier()

    pltpu.sync_copy(src_hbm.at[pl.ds(gbase, 1)], src_idx.at[pl.ds(0, 1)])
    pltpu.sync_copy(dst_hbm.at[pl.ds(gbase, 1)], dst_idx.at[pl.ds(0, 1)])
    for b in range(_GROUP - 1):
        pltpu.async_copy(y_hbm.at[src_idx.at[0, b]], bufs.at[b], gsem)

    def body(i, carry):
        h = lax.rem(i, 2)
        not_last = i < _GROUPS_PER_W - 1
        for b in range(_GROUP):
            pltpu.make_async_copy(
                y_hbm.at[src_idx.at[h, b]], bufs.at[b], gsem).wait()
            pltpu.async_copy(bufs.at[b], acc.at[dst_idx.at[h, b]], ssem,
                             add=True)

            def _wait_prev_scatter(b=b, h=h):
                pltpu.make_async_copy(
                    bufs.at[(b + _GROUP - 1) % _GROUP],
                    acc.at[dst_idx.at[h, b]], ssem).wait()

            if b == 0:
                pl.when(i >= 1)(_wait_prev_scatter)

                @pl.when(not_last)
                def _():
                    pltpu.async_copy(src_hbm.at[pl.ds(gbase + i + 1, 1)],
                                     src_idx.at[pl.ds(1 - h, 1)], isem)
                    pltpu.async_copy(dst_hbm.at[pl.ds(gbase + i + 1, 1)],
                                     dst_idx.at[pl.ds(1 - h, 1)], isem)

                pltpu.async_copy(y_hbm.at[src_idx.at[h, _GROUP - 1]],
                                 bufs.at[_GROUP - 1], gsem)
            elif b == 1:
                _wait_prev_scatter()

                @pl.when(not_last)
                def _():
                    pltpu.make_async_copy(
                        src_hbm.at[pl.ds(gbase + i + 1, 1)],
                        src_idx.at[pl.ds(1 - h, 1)], isem).wait()
                    pltpu.make_async_copy(
                        dst_hbm.at[pl.ds(gbase + i + 1, 1)],
                        dst_idx.at[pl.ds(1 - h, 1)], isem).wait()
                    pltpu.async_copy(
                        y_hbm.at[src_idx.at[1 - h, 0]], bufs.at[0], gsem)
            else:
                _wait_prev_scatter()

                @pl.when(not_last)
                def _(b=b):
                    pltpu.async_copy(
                        y_hbm.at[src_idx.at[1 - h, b - 1]], bufs.at[b - 1],
                        gsem)
        return carry

    lax.fori_loop(0, _GROUPS_PER_W, body, 0)
    pltpu.make_async_copy(bufs.at[_GROUP - 1], acc.at[dst_idx.at[0, _GROUP - 1]],
                          ssem).wait()

    plsc.subcore_barrier()
    out_off = c * _N + s * _ROWS_PER_TILE_OUT
    pltpu.sync_copy(
        acc.at[pl.ds(s * _ROWS_PER_TILE_OUT, _ROWS_PER_TILE_OUT)],
        out_hbm.at[pl.ds(out_off, _ROWS_PER_TILE_OUT)],
    )

    tail_rows = _N - _NS * _ROWS_PER_TILE_OUT

    @pl.when(s == _NS - 1)
    def _():
        pltpu.sync_copy(
            acc.at[pl.ds(_NS * _ROWS_PER_TILE_OUT, tail_rows)],
            out_hbm.at[pl.ds(c * _N + _NS * _ROWS_PER_TILE_OUT, tail_rows)],
        )


def kernel(x, edge_index, W1, b1, W2, b2, W3, b3):
    src = edge_index[0]
    dst = edge_index[1]
    n_edges = src.shape[0]
    pad = _E_PAD - n_edges

    pad_idx = jnp.arange(pad, dtype=jnp.int32)
    src_g = jnp.concatenate([src, pad_idx]).reshape(_N_GROUPS, _GROUP, _CHUNK)
    dst_g = jnp.concatenate([dst, pad_idx]).reshape(_N_GROUPS, _GROUP, _CHUNK)
    zeros = jnp.zeros((_N, _D), jnp.float32)

    t = _tc_pre(x)
    parts = _sc_segsum(t, src_g, dst_g, zeros)
    t = _tc_post_mid(parts, parts, t, W1, b1.reshape(1, _D))
    parts = _sc_segsum(t, src_g, dst_g, zeros)
    t = _tc_post_mid(parts, parts, t, W2, b2.reshape(1, _D))
    parts = _sc_segsum(t, src_g, dst_g, zeros)
    return _tc_post_final(parts, parts, t, W3, b3.reshape(1, _D))


# speedup vs baseline: 1.0166x; 1.0166x over previous
import functools

import jax
import jax.numpy as jnp
import numpy as np
from jax import lax
from jax.experimental import pallas as pl
from jax.experimental.pallas import tpu as pltpu
from jax.experimental.pallas import tpu_sc as plsc


def _logmap0(x):
    norm = jnp.sqrt(jnp.sum(x * x, axis=-1, keepdims=True))
    norm_c = jnp.maximum(norm, 1e-10)
    arg = jnp.minimum(norm_c, 1.0 - 1e-5)
    atanh = 0.5 * jnp.log((1.0 + arg) / (1.0 - arg))
    return x * (atanh / norm_c)


def _expmap0(u):
    norm = jnp.sqrt(jnp.sum(u * u, axis=-1, keepdims=True))
    norm_c = jnp.maximum(norm, 1e-10)
    return jnp.tanh(norm_c) * u / norm_c


_ATANH_MAX = float(np.arctanh(np.float64(1.0 - 1e-5)))


def _norm_clip(u):
    norm = jnp.sqrt(jnp.sum(u * u, axis=-1, keepdims=True))
    norm_c = jnp.maximum(norm, 1e-10)
    return u * (jnp.minimum(norm, _ATANH_MAX) / norm_c)


_N = 10000
_D = 128
_ROWS_PER_BLOCK = 5000
_N_BLOCKS = _N // _ROWS_PER_BLOCK


def _tc_pre_body(x_ref, o_ref):
    o_ref[...] = _logmap0(x_ref[...])


_tc_pre = pl.pallas_call(
    _tc_pre_body,
    grid=(_N_BLOCKS,),
    in_specs=[pl.BlockSpec((_ROWS_PER_BLOCK, _D), lambda i: (i, 0))],
    out_specs=pl.BlockSpec((_ROWS_PER_BLOCK, _D), lambda i: (i, 0)),
    out_shape=jax.ShapeDtypeStruct((_N, _D), jnp.float32),
)


def _tc_post_body(p0_ref, p1_ref, t_ref, w_ref, b_ref, o_ref, *, mode):
    i = pl.program_id(0)
    rows = i * _ROWS_PER_BLOCK + lax.broadcasted_iota(
        jnp.int32, (_ROWS_PER_BLOCK, 1), 0)
    pad_fix = jnp.where(rows < _PAD_EDGES, t_ref[...], 0.0)
    s = p0_ref[...] + p1_ref[...] - pad_fix
    o = lax.dot_general(
        s, w_ref[...], (((1,), (1,)), ((), ())),
        preferred_element_type=jnp.float32,
    ) + b_ref[...]
    if mode == "mid":
        h = _norm_clip(jnp.maximum(_norm_clip(o), 0.0))
    else:
        h = _expmap0(o)
    o_ref[...] = h


def _make_tc_post(mode):
    return pl.pallas_call(
        functools.partial(_tc_post_body, mode=mode),
        grid=(_N_BLOCKS,),
        in_specs=[
            pl.BlockSpec((_ROWS_PER_BLOCK, _D), lambda i: (i, 0)),
            pl.BlockSpec((_ROWS_PER_BLOCK, _D), lambda i: (i + _N_BLOCKS, 0)),
            pl.BlockSpec((_ROWS_PER_BLOCK, _D), lambda i: (i, 0)),
            pl.BlockSpec((_D, _D), lambda i: (0, 0)),
            pl.BlockSpec((1, _D), lambda i: (0, 0)),
        ],
        out_specs=pl.BlockSpec((_ROWS_PER_BLOCK, _D), lambda i: (i, 0)),
        out_shape=jax.ShapeDtypeStruct((_N, _D), jnp.float32),
    )


_tc_post_mid = _make_tc_post("mid")
_tc_post_final = _make_tc_post("final")


_NC = 2
_NS = 16
_NW = _NC * _NS
_CHUNK = 120
_GROUP = 3
_GROUPS_PER_W = 28
_CHUNKS_PER_W = _GROUPS_PER_W * _GROUP
_E_PAD = _NW * _CHUNKS_PER_W * _CHUNK
_N_GROUPS = _NW * _GROUPS_PER_W
_PAD_EDGES = _E_PAD - 320000
_ROWS_PER_TILE_OUT = 624

_sc_mesh = plsc.VectorSubcoreMesh(core_axis_name="c", subcore_axis_name="s")


@functools.partial(
    pl.kernel,
    out_type=jax.ShapeDtypeStruct((_NC * _N, _D), jnp.float32),
    mesh=_sc_mesh,
    scratch_types=[
        pltpu.VMEM((2, _GROUP, _CHUNK), jnp.int32),
        pltpu.VMEM((2, _GROUP, _CHUNK), jnp.int32),
        pltpu.VMEM((_GROUP, _CHUNK, _D), jnp.float32),
        pltpu.VMEM_SHARED((_N, _D), jnp.float32),
        pltpu.SemaphoreType.DMA,
        pltpu.SemaphoreType.DMA,
        pltpu.SemaphoreType.DMA,
    ],
)
def _sc_segsum(y_hbm, src_hbm, dst_hbm, zeros_hbm, out_hbm,
               src_idx, dst_idx, bufs, acc, gsem, ssem, isem):
    c = lax.axis_index("c")
    s = lax.axis_index("s")
    wid = c * _NS + s
    gbase = wid * _GROUPS_PER_W

    @pl.when(s < _NS - 1)
    def _():
        pltpu.sync_copy(zeros_hbm.at[pl.ds(s * 632, 632)],
                        acc.at[pl.ds(s * 632, 632)])

    @pl.when(s == _NS - 1)
    def _():
        pltpu.sync_copy(zeros_hbm.at[pl.ds(9480, 520)],
                        acc.at[pl.ds(9480, 520)])

    plsc.subcore_barrier()

    pltpu.sync_copy(src_hbm.at[pl.ds(gbase, 1)], src_idx.at[pl.ds(0, 1)])
    pltpu.sync_copy(dst_hbm.at[pl.ds(gbase, 1)], dst_idx.at[pl.ds(0, 1)])
    for b in range(_GROUP - 1):
        pltpu.async_copy(y_hbm.at[src_idx.at[0, b]], bufs.at[b], gsem)

    def body(i, carry):
        h = lax.rem(i, 2)
        not_last = i < _GROUPS_PER_W - 1
        for b in range(_GROUP):
            pltpu.make_async_copy(
                y_hbm.at[src_idx.at[h, b]], bufs.at[b], gsem).wait()
            pltpu.async_copy(bufs.at[b], acc.at[dst_idx.at[h, b]], ssem,
                             add=True)

            def _wait_prev_scatter(b=b, h=h):
                pltpu.make_async_copy(
                    bufs.at[(b + _GROUP - 1) % _GROUP],
                    acc.at[dst_idx.at[h, b]], ssem).wait()

            if b == 0:
                pl.when(i >= 1)(_wait_prev_scatter)

                @pl.when(not_last)
                def _():
                    pltpu.async_copy(src_hbm.at[pl.ds(gbase + i + 1, 1)],
                                     src_idx.at[pl.ds(1 - h, 1)], isem)
                    pltpu.async_copy(dst_hbm.at[pl.ds(gbase + i + 1, 1)],
                                     dst_idx.at[pl.ds(1 - h, 1)], isem)

                pltpu.async_copy(y_hbm.at[src_idx.at[h, _GROUP - 1]],
                                 bufs.at[_GROUP - 1], gsem)
            elif b == 1:
                _wait_prev_scatter()

                @pl.when(not_last)
                def _():
                    pltpu.make_async_copy(
                        src_hbm.at[pl.ds(gbase + i + 1, 1)],
                        src_idx.at[pl.ds(1 - h, 1)], isem).wait()
                    pltpu.make_async_copy(
                        dst_hbm.at[pl.ds(gbase + i + 1, 1)],
                        dst_idx.at[pl.ds(1 - h, 1)], isem).wait()
                    pltpu.async_copy(
                        y_hbm.at[src_idx.at[1 - h, 0]], bufs.at[0], gsem)
            else:
                _wait_prev_scatter()

                @pl.when(not_last)
                def _(b=b):
                    pltpu.async_copy(
                        y_hbm.at[src_idx.at[1 - h, b - 1]], bufs.at[b - 1],
                        gsem)
        return carry

    lax.fori_loop(0, _GROUPS_PER_W, body, 0)
    pltpu.make_async_copy(bufs.at[_GROUP - 1], acc.at[dst_idx.at[0, _GROUP - 1]],
                          ssem).wait()

    plsc.subcore_barrier()
    out_off = c * _N + s * _ROWS_PER_TILE_OUT
    pltpu.sync_copy(
        acc.at[pl.ds(s * _ROWS_PER_TILE_OUT, _ROWS_PER_TILE_OUT)],
        out_hbm.at[pl.ds(out_off, _ROWS_PER_TILE_OUT)],
    )

    tail_rows = _N - _NS * _ROWS_PER_TILE_OUT

    @pl.when(s == _NS - 1)
    def _():
        pltpu.sync_copy(
            acc.at[pl.ds(_NS * _ROWS_PER_TILE_OUT, tail_rows)],
            out_hbm.at[pl.ds(c * _N + _NS * _ROWS_PER_TILE_OUT, tail_rows)],
        )


def kernel(x, edge_index, W1, b1, W2, b2, W3, b3):
    src = edge_index[0]
    dst = edge_index[1]
    n_edges = src.shape[0]
    pad = _E_PAD - n_edges

    pad_idx = jnp.arange(pad, dtype=jnp.int32)
    src_g = jnp.concatenate([src, pad_idx]).reshape(_N_GROUPS, _GROUP, _CHUNK)
    dst_g = jnp.concatenate([dst, pad_idx]).reshape(_N_GROUPS, _GROUP, _CHUNK)
    zeros = jnp.zeros((_N, _D), jnp.float32)

    t = _tc_pre(x)
    parts = _sc_segsum(t, src_g, dst_g, zeros)
    t = _tc_post_mid(parts, parts, t, W1, b1.reshape(1, _D))
    parts = _sc_segsum(t, src_g, dst_g, zeros)
    t = _tc_post_mid(parts, parts, t, W2, b2.reshape(1, _D))
    parts = _sc_segsum(t, src_g, dst_g, zeros)
    return _tc_post_final(parts, parts, t, W3, b3.reshape(1, _D))
